# Initial kernel scaffold; baseline (speedup 1.0000x reference)
import jax
import jax.numpy as jnp
from jax.experimental import pallas as pl


def kernel(x, y):
    raise NotImplementedError("write your pallas kernel here")


# speedup vs baseline: 2.8844x; 2.8844x over previous
import jax
import jax.numpy as jnp
from jax import lax
from jax.experimental import pallas as pl
from jax.experimental.pallas import tpu as pltpu
from jax.experimental.pallas import tpu_sc as plsc

_N_BINS = 256
_N_ELEM = 2 * 3 * 224 * 224
_NC = 2
_NS = 16
_NW = _NC * _NS
_CHUNK = _N_ELEM // _NW
_VREGS = _CHUNK // 16
_COLS = 288
_HWORDS = 16 * _COLS
_SCALE = 1.0 / (float(_N_ELEM) ** 2 * float(_N_BINS))


def _sc_body(x_hbm, y_hbm, outx_hbm, outy_hbm, buf, hx, hy):
    wid = lax.axis_index("s") * _NC + lax.axis_index("c")
    zero = jnp.zeros((16,), jnp.float32)

    def zero_body(j, c):
        hx[pl.ds(j * 16, 16)] = zero
        hy[pl.ds(j * 16, 16)] = zero
        return c

    lax.fori_loop(0, _COLS, zero_body, 0)

    lane_off = lax.iota(jnp.int32, 16) * _COLS

    def accum(h_ref):
        def body(i, c):
            v = buf[pl.ds(i * 16, 16)]
            u = jnp.minimum(jnp.maximum(v * 255.0, 0.0), 255.0)
            b0 = u.astype(jnp.int32)
            f = u - b0.astype(jnp.float32)
            idx = lane_off + b0
            plsc.addupdate_scatter(h_ref, [idx], 1.0 - f)
            plsc.addupdate_scatter(h_ref, [idx + 1], f)
            return c

        lax.fori_loop(0, _VREGS, body, 0)

    base = wid * _CHUNK
    pltpu.sync_copy(x_hbm.at[pl.ds(base, _CHUNK)], buf)
    accum(hx)
    pltpu.sync_copy(y_hbm.at[pl.ds(base, _CHUNK)], buf)
    accum(hy)

    pltpu.sync_copy(hx, outx_hbm.at[wid])
    pltpu.sync_copy(hy, outy_hbm.at[wid])


def _sc_hist(xf, yf):
    mesh = plsc.VectorSubcoreMesh(core_axis_name="c", subcore_axis_name="s")
    part = jax.ShapeDtypeStruct((_NW, _HWORDS), jnp.float32)
    f = pl.kernel(
        _sc_body,
        out_type=[part, part],
        mesh=mesh,
        compiler_params=pltpu.CompilerParams(needs_layout_passes=False),
        scratch_types=[
            pltpu.VMEM((_CHUNK,), jnp.float32),
            pltpu.VMEM((_HWORDS,), jnp.float32),
            pltpu.VMEM((_HWORDS,), jnp.float32),
        ],
    )
    return f(xf, yf)


def _tc_loss_body(hx_ref, hy_ref, o_ref):
    d = jnp.sum(hx_ref[...] - hy_ref[...], axis=0, keepdims=True)
    s = jnp.sum(d * d) * _SCALE
    o_ref[...] = jnp.reshape(s, (1, 1))


def _tc_loss(hxp, hyp):
    return pl.pallas_call(
        _tc_loss_body,
        out_shape=jax.ShapeDtypeStruct((1, 1), jnp.float32),
    )(hxp, hyp)


def kernel(x, y):
    xf = x.reshape(-1)
    yf = y.reshape(-1)
    hxp, hyp = _sc_hist(xf, yf)
    hxp = hxp.reshape(_NW * 16, _COLS)
    hyp = hyp.reshape(_NW * 16, _COLS)
    return _tc_loss(hxp, hyp)[0, 0]


# speedup vs baseline: 4.2018x; 1.4567x over previous
import jax
import jax.numpy as jnp
from jax import lax
from jax.experimental import pallas as pl
from jax.experimental.pallas import tpu as pltpu
from jax.experimental.pallas import tpu_sc as plsc

_N_BINS = 256
_N_ELEM = 2 * 3 * 224 * 224
_NC = 2
_NS = 16
_NW = _NC * _NS
_CHUNK = _N_ELEM // _NW
_COLS = 272
_HWORDS = 16 * _COLS
_SCALE = 1.0 / (float(_N_ELEM) ** 2 * float(_N_BINS))


def _sc_body(x_hbm, y_hbm, outx_hbm, outy_hbm, bufx, bufy, hx, hy, semx, semy):
    wid = lax.axis_index("s") * _NC + lax.axis_index("c")
    base = wid * _CHUNK
    cpx = pltpu.async_copy(x_hbm.at[pl.ds(base, _CHUNK)], bufx, semx)
    cpy = pltpu.async_copy(y_hbm.at[pl.ds(base, _CHUNK)], bufy, semy)

    zero = jnp.zeros((16,), jnp.float32)

    @plsc.parallel_loop(0, _HWORDS, step=16, unroll=8)
    def _zero(o):
        hx[pl.ds(o, 16)] = zero
        hy[pl.ds(o, 16)] = zero

    cpx.wait()
    cpy.wait()

    lane_off = lax.iota(jnp.int32, 16) * _COLS
    one = jnp.float32(1.0)

    @plsc.parallel_loop(0, _CHUNK, step=16, unroll=4)
    def _accum(o):
        vx = bufx[pl.ds(o, 16)]
        ux = jnp.minimum(jnp.maximum(vx * 255.0, 0.0), 255.0)
        bx = ux.astype(jnp.int32)
        fx = ux - bx.astype(jnp.float32)
        ix = lane_off + bx
        plsc.addupdate_scatter(hx, [ix], one - fx)
        plsc.addupdate_scatter(hx, [ix + 1], fx)
        vy = bufy[pl.ds(o, 16)]
        uy = jnp.minimum(jnp.maximum(vy * 255.0, 0.0), 255.0)
        by = uy.astype(jnp.int32)
        fy = uy - by.astype(jnp.float32)
        iy = lane_off + by
        plsc.addupdate_scatter(hy, [iy], one - fy)
        plsc.addupdate_scatter(hy, [iy + 1], fy)

    pltpu.sync_copy(hx, outx_hbm.at[wid])
    pltpu.sync_copy(hy, outy_hbm.at[wid])


def _sc_hist(xf, yf):
    mesh = plsc.VectorSubcoreMesh(core_axis_name="c", subcore_axis_name="s")
    part = jax.ShapeDtypeStruct((_NW, _HWORDS), jnp.float32)
    f = pl.kernel(
        _sc_body,
        out_type=[part, part],
        mesh=mesh,
        compiler_params=pltpu.CompilerParams(needs_layout_passes=False),
        scratch_types=[
            pltpu.VMEM((_CHUNK,), jnp.float32),
            pltpu.VMEM((_CHUNK,), jnp.float32),
            pltpu.VMEM((_HWORDS,), jnp.float32),
            pltpu.VMEM((_HWORDS,), jnp.float32),
            pltpu.SemaphoreType.DMA,
            pltpu.SemaphoreType.DMA,
        ],
    )
    return f(xf, yf)


def _tc_loss_body(hx_ref, hy_ref, o_ref):
    d = jnp.sum(hx_ref[...] - hy_ref[...], axis=0, keepdims=True)
    s = jnp.sum(d * d) * _SCALE
    o_ref[...] = jnp.reshape(s, (1, 1))


def _tc_loss(hxp, hyp):
    return pl.pallas_call(
        _tc_loss_body,
        out_shape=jax.ShapeDtypeStruct((1, 1), jnp.float32),
    )(hxp, hyp)


def kernel(x, y):
    xf = x.reshape(-1)
    yf = y.reshape(-1)
    hxp, hyp = _sc_hist(xf, yf)
    hxp = hxp.reshape(_NW * 16, _COLS)
    hyp = hyp.reshape(_NW * 16, _COLS)
    return _tc_loss(hxp, hyp)[0, 0]


# speedup vs baseline: 4.8187x; 1.1468x over previous
import jax
import jax.numpy as jnp
from jax import lax
from jax.experimental import pallas as pl
from jax.experimental.pallas import tpu as pltpu
from jax.experimental.pallas import tpu_sc as plsc

_N_BINS = 256
_N_ELEM = 2 * 3 * 224 * 224
_NC = 2
_NS = 16
_NW = _NC * _NS
_CHUNK = _N_ELEM // _NW
_COLS = _N_BINS
_HWORDS = 16 * _COLS + 16
_SCALE = 1.0 / (float(_N_ELEM) ** 2 * float(_N_BINS))


def _sc_body(x_hbm, y_hbm, outx_hbm, outy_hbm, bufx, bufy, hx, hy, hxr, hyr,
             semx, semy):
    wid = lax.axis_index("s") * _NC + lax.axis_index("c")
    base = wid * _CHUNK
    cpx = pltpu.async_copy(x_hbm.at[pl.ds(base, _CHUNK)], bufx, semx)
    cpy = pltpu.async_copy(y_hbm.at[pl.ds(base, _CHUNK)], bufy, semy)

    zero = jnp.zeros((16,), jnp.float32)

    @plsc.parallel_loop(0, _HWORDS, step=16, unroll=8)
    def _zero(o):
        hx[pl.ds(o, 16)] = zero
        hy[pl.ds(o, 16)] = zero

    cpx.wait()
    cpy.wait()

    lane_off = lax.iota(jnp.int32, 16) * _COLS
    one = jnp.float32(1.0)

    @plsc.parallel_loop(0, _CHUNK, step=16, unroll=4)
    def _accum(o):
        vx = bufx[pl.ds(o, 16)]
        ux = jnp.minimum(jnp.maximum(vx * 255.0, 0.0), 255.0)
        bx = ux.astype(jnp.int32)
        fx = ux - bx.astype(jnp.float32)
        ix = lane_off + bx
        plsc.addupdate_scatter(hx, [ix], one - fx)
        plsc.addupdate_scatter(hx, [ix + 1], fx)
        vy = bufy[pl.ds(o, 16)]
        uy = jnp.minimum(jnp.maximum(vy * 255.0, 0.0), 255.0)
        by = uy.astype(jnp.int32)
        fy = uy - by.astype(jnp.float32)
        iy = lane_off + by
        plsc.addupdate_scatter(hy, [iy], one - fy)
        plsc.addupdate_scatter(hy, [iy + 1], fy)

    @plsc.parallel_loop(0, _N_BINS, step=16, unroll=2)
    def _fold(j):
        ax = hx[pl.ds(j, 16)]
        ay = hy[pl.ds(j, 16)]
        for l in range(1, 16):
            ax = ax + hx[pl.ds(l * _COLS + j, 16)]
            ay = ay + hy[pl.ds(l * _COLS + j, 16)]
        hxr[pl.ds(j, 16)] = ax
        hyr[pl.ds(j, 16)] = ay

    pltpu.sync_copy(hxr, outx_hbm.at[wid])
    pltpu.sync_copy(hyr, outy_hbm.at[wid])


def _sc_hist(xf, yf):
    mesh = plsc.VectorSubcoreMesh(core_axis_name="c", subcore_axis_name="s")
    part = jax.ShapeDtypeStruct((_NW, _N_BINS), jnp.float32)
    f = pl.kernel(
        _sc_body,
        out_type=[part, part],
        mesh=mesh,
        compiler_params=pltpu.CompilerParams(needs_layout_passes=False),
        scratch_types=[
            pltpu.VMEM((_CHUNK,), jnp.float32),
            pltpu.VMEM((_CHUNK,), jnp.float32),
            pltpu.VMEM((_HWORDS,), jnp.float32),
            pltpu.VMEM((_HWORDS,), jnp.float32),
            pltpu.VMEM((_N_BINS,), jnp.float32),
            pltpu.VMEM((_N_BINS,), jnp.float32),
            pltpu.SemaphoreType.DMA,
            pltpu.SemaphoreType.DMA,
        ],
    )
    return f(xf, yf)


def _tc_loss_body(hx_ref, hy_ref, o_ref):
    d = jnp.sum(hx_ref[...] - hy_ref[...], axis=0, keepdims=True)
    s = jnp.sum(d * d) * _SCALE
    o_ref[...] = jnp.reshape(s, (1, 1))


def _tc_loss(hxp, hyp):
    return pl.pallas_call(
        _tc_loss_body,
        out_shape=jax.ShapeDtypeStruct((1, 1), jnp.float32),
    )(hxp, hyp)


def kernel(x, y):
    xf = x.reshape(-1)
    yf = y.reshape(-1)
    hxp, hyp = _sc_hist(xf, yf)
    return _tc_loss(hxp, hyp)[0, 0]
